# baseline (speedup 1.0000x reference)
import functools

import jax
import jax.numpy as jnp
from jax import lax
from jax.experimental import pallas as pl
from jax.experimental.pallas import tpu as pltpu
from jax.experimental.pallas import tpu_sc as plsc

VOCAB = 100000
EMB = 300
HID = 4096
B = 4096
L = 200

NC = 2
NS = 16
NW = NC * NS
NCHUNK = 4
BC = B // NCHUNK
BPW = BC // NW
LA = 104
LB = L - LA

EMBP = 384
HALF = EMBP // 2
PACKW = 256

_NCHP = HALF // 16


NU = 2 * BPW


def _sc_pool_body(x_hbm, emb_hbm, out_hbm, *refs):
    idxb = refs[0:4]
    bufs = refs[4:8]
    out_v = refs[8]
    gsem = refs[9:13]
    isem = refs[13:17]

    wid = lax.axis_index("s") * NC + lax.axis_index("c")
    base = wid * BPW
    zero = jnp.zeros((16,), jnp.float32)
    scale = jnp.full((16,), 1.0 / L, jnp.float32)

    def fire_idx(r, slot):
        pltpu.async_copy(x_hbm.at[pl.ds((base + r) * L, L)],
                         idxb[slot], isem[slot])

    def wait_idx(r, slot):
        pltpu.make_async_copy(x_hbm.at[pl.ds((base + r) * L, L)],
                              idxb[slot], isem[slot]).wait()

    def gather(islot, half, bslot):
        if half == 0:
            src = emb_hbm.at[idxb[islot].at[pl.ds(0, LA)]]
            dst = bufs[bslot]
        else:
            src = emb_hbm.at[idxb[islot].at[pl.ds(LA, LB)]]
            dst = bufs[bslot].at[pl.ds(0, LB)]
        return pltpu.make_async_copy(src, dst, gsem[bslot])

    def reduce_rows(buf, n):
        @pl.loop(0, n, init_carry=(zero,) * (2 * _NCHP))
        def sums(r, carry):
            vals = []
            for c in range(_NCHP):
                w = buf[r, pl.ds(16 * c, 16)]
                lo, hi = plsc.unpack(plsc.bitcast(w, jnp.bfloat16),
                                     format=plsc.PackFormat.INTERLEAVED)
                vals.append(carry[2 * c] + lo)
                vals.append(carry[2 * c + 1] + hi)
            return tuple(vals)
        return sums

    fire_idx(0, 0)
    fire_idx(1, 1)
    fire_idx(2, 2)
    wait_idx(0, 0)
    gather(0, 0, 0).start()
    gather(0, 1, 1).start()
    wait_idx(1, 1)
    gather(1, 0, 2).start()

    @pl.loop(0, NU // 8)
    def _(t):
        for j in range(8):
            u = t * 8 + j
            r = u // 2
            rs = (j // 2) % 4
            gather(rs, j % 2, j % 4).wait()

            @pl.when(u + 3 < NU)
            def _():
                if j % 2 == 0:
                    gather((rs + 1) % 4, 1, (j + 3) % 4).start()
                else:
                    wait_idx(r + 2, (rs + 2) % 4)
                    gather((rs + 2) % 4, 0, (j + 3) % 4).start()

            if j % 2 == 1:
                @pl.when(r + 3 < BPW)
                def _():
                    fire_idx(r + 3, (rs + 3) % 4)

            if j % 2 == 0:
                sums = reduce_rows(bufs[j % 4], LA)
                for c in range(_NCHP):
                    out_v[r, pl.ds(16 * c, 16)] = sums[2 * c]
                    out_v[r, pl.ds(HALF + 16 * c, 16)] = sums[2 * c + 1]
            else:
                sums = reduce_rows(bufs[j % 4], LB)
                for c in range(_NCHP):
                    out_v[r, pl.ds(16 * c, 16)] = (
                        out_v[r, pl.ds(16 * c, 16)] + sums[2 * c]) * scale
                    out_v[r, pl.ds(HALF + 16 * c, 16)] = (
                        out_v[r, pl.ds(HALF + 16 * c, 16)]
                        + sums[2 * c + 1]) * scale

    pltpu.sync_copy(out_v, out_hbm.at[pl.ds(base, BPW)])


_sc_pool = functools.partial(
    pl.kernel,
    out_type=jax.ShapeDtypeStruct((BC, EMBP), jnp.float32),
    mesh=plsc.VectorSubcoreMesh(core_axis_name="c", subcore_axis_name="s"),
    scratch_types=(
        [pltpu.VMEM((L,), jnp.int32) for _ in range(4)]
        + [pltpu.VMEM((LA, PACKW), jnp.float32) for _ in range(4)]
        + [pltpu.VMEM((BPW, EMBP), jnp.float32)]
        + [pltpu.SemaphoreType.DMA for _ in range(8)]
    ),
    compiler_params=pltpu.CompilerParams(needs_layout_passes=False),
)(_sc_pool_body)


def _trans_body(a_ref, i_ref, o_ref):
    res = lax.dot_general(
        a_ref[...].astype(jnp.bfloat16), i_ref[...],
        dimension_numbers=(((0,), (0,)), ((), ())),
        preferred_element_type=jnp.float32)
    bits = lax.bitcast_convert_type(res, jnp.uint32) + jnp.uint32(0x8000)
    lo = bits[:, :HALF] >> 16
    hi = bits[:, HALF:] & jnp.uint32(0xFFFF0000)
    o_ref[:, :HALF] = lax.bitcast_convert_type(lo | hi, jnp.float32)
    o_ref[:, HALF:] = jnp.zeros((o_ref.shape[0], PACKW - HALF), jnp.float32)


_TV = 2048

_trans = pl.pallas_call(
    _trans_body,
    grid=((VOCAB + _TV - 1) // _TV,),
    in_specs=[
        pl.BlockSpec((EMB, _TV), lambda i: (0, i)),
        pl.BlockSpec((EMB, EMBP), lambda i: (0, 0)),
    ],
    out_specs=pl.BlockSpec((_TV, PACKW), lambda i: (i, 0)),
    out_shape=jax.ShapeDtypeStruct((VOCAB, PACKW), jnp.float32),
    compiler_params=pltpu.CompilerParams(
        dimension_semantics=("arbitrary",)),
)


def _mlp_body(x_ref, w1_ref, b1_ref, w2_ref, b2_ref, w3_ref, b3_ref, o_ref):
    h = jnp.dot(x_ref[...], w1_ref[...], preferred_element_type=jnp.float32)
    h = jnp.maximum(h + b1_ref[...], 0.0).astype(jnp.bfloat16)
    h = jnp.dot(h, w2_ref[...], preferred_element_type=jnp.float32)
    h = jnp.maximum(h + b2_ref[...], 0.0).astype(jnp.bfloat16)
    logits = jnp.dot(h, w3_ref[...], preferred_element_type=jnp.float32)
    logits = logits + b3_ref[...]
    m = jnp.max(logits, axis=1, keepdims=True)
    lse = jnp.log(jnp.sum(jnp.exp(logits - m), axis=1, keepdims=True)) + m
    o_ref[...] = logits - lse


BM = 512

_mlp = pl.pallas_call(
    _mlp_body,
    grid=(BC // BM,),
    in_specs=[
        pl.BlockSpec((BM, EMBP), lambda i: (i, 0)),
        pl.BlockSpec((EMBP, HID), lambda i: (0, 0)),
        pl.BlockSpec((1, HID), lambda i: (0, 0)),
        pl.BlockSpec((HID, HID), lambda i: (0, 0)),
        pl.BlockSpec((1, HID), lambda i: (0, 0)),
        pl.BlockSpec((HID, 2), lambda i: (0, 0)),
        pl.BlockSpec((1, 2), lambda i: (0, 0)),
    ],
    out_specs=pl.BlockSpec((BM, 2), lambda i: (i, 0)),
    out_shape=jax.ShapeDtypeStruct((BC, 2), jnp.float32),
    compiler_params=pltpu.CompilerParams(
        dimension_semantics=("arbitrary",)),
)


def kernel(x, emb, W1, b1, W2, b2, W3, b3):
    emb_p = _trans(emb.T, jnp.eye(EMB, EMBP, dtype=jnp.bfloat16))
    w1_p = jnp.pad(W1, ((0, EMBP - EMB), (0, 0)))
    w1b = w1_p.astype(jnp.bfloat16)
    w2b = W2.astype(jnp.bfloat16)
    w3b = W3.astype(jnp.bfloat16)
    b1r = b1.reshape(1, HID)
    b2r = b2.reshape(1, HID)
    b3r = b3.reshape(1, 2)
    xf = x.reshape(-1)
    pooled = [_sc_pool(lax.dynamic_slice_in_dim(xf, k * BC * L, BC * L),
                       emb_p)
              for k in range(NCHUNK)]
    outs = [_mlp(p.astype(jnp.bfloat16), w1b, b1r, w2b, b2r, w3b, b3r)
            for p in pooled]
    return jnp.concatenate(outs, axis=0)
